# Initial kernel scaffold; baseline (speedup 1.0000x reference)
import functools

import jax
import jax.numpy as jnp
from jax import lax
from jax.experimental import pallas as pl
from jax.experimental.pallas import tpu as pltpu
from jax.experimental.pallas import tpu_sc as plsc

N = 10000
E = 320000
F_IN = 128
H = 64
C = 6
G = 64

NC = 2
NS = 16
NW = NC * NS
K = 80
EPW = E // NW
NCHUNK = EPW // K
RPS = N // NS

_mesh = plsc.VectorSubcoreMesh(core_axis_name="c", subcore_axis_name="s")


@functools.partial(
    pl.kernel,
    out_type=jax.ShapeDtypeStruct((NC * N, 16), jnp.float32),
    mesh=_mesh,
    scratch_types=[
        pltpu.VMEM((K,), jnp.int32),
        pltpu.VMEM((K, 16), jnp.float32),
        pltpu.VMEM_SHARED((N, 16), jnp.float32),
    ],
)
def _sc_count(dst_hbm, zeros_hbm, ones_hbm, out_hbm, didx, ones_v, acc):
    cid = lax.axis_index("c")
    sid = lax.axis_index("s")
    wid = sid * NC + cid
    pltpu.sync_copy(zeros_hbm.at[pl.ds(sid * RPS, RPS)],
                    acc.at[pl.ds(sid * RPS, RPS)])
    pltpu.sync_copy(ones_hbm, ones_v)
    plsc.subcore_barrier()
    base = wid * EPW

    @pl.loop(0, NCHUNK)
    def _(c):
        pltpu.sync_copy(dst_hbm.at[pl.ds(base + c * K, K)], didx)
        pltpu.sync_copy(ones_v, acc.at[didx], add=True)

    plsc.subcore_barrier()
    pltpu.sync_copy(acc.at[pl.ds(sid * RPS, RPS)],
                    out_hbm.at[pl.ds(cid * N + sid * RPS, RPS)])


@functools.partial(
    pl.kernel,
    out_type=jax.ShapeDtypeStruct((NC * N, H), jnp.float32),
    mesh=_mesh,
    scratch_types=[
        pltpu.VMEM((K,), jnp.int32),
        pltpu.VMEM((K,), jnp.int32),
        pltpu.VMEM((K, H), jnp.float32),
        pltpu.VMEM_SHARED((N, H), jnp.float32),
        pltpu.SemaphoreType.DMA,
    ],
)
def _sc_agg(hs_hbm, src_hbm, dst_hbm, zeros_hbm, out_hbm,
            sidx, didx, rows, acc, sem):
    cid = lax.axis_index("c")
    sid = lax.axis_index("s")
    wid = sid * NC + cid
    pltpu.sync_copy(zeros_hbm.at[pl.ds(sid * RPS, RPS)],
                    acc.at[pl.ds(sid * RPS, RPS)])
    plsc.subcore_barrier()
    base = wid * EPW

    @pl.loop(0, NCHUNK)
    def _(c):
        pltpu.sync_copy(src_hbm.at[pl.ds(base + c * K, K)], sidx)
        pltpu.sync_copy(dst_hbm.at[pl.ds(base + c * K, K)], didx)
        pltpu.async_copy(hs_hbm.at[sidx], rows, sem).wait()
        pltpu.sync_copy(rows, acc.at[didx], add=True)

    plsc.subcore_barrier()
    pltpu.sync_copy(acc.at[pl.ds(sid * RPS, RPS)],
                    out_hbm.at[pl.ds(cid * N + sid * RPS, RPS)])


def _tc_prep_body(x_ref, w1_ref, cnt_ref, hs_ref, dinv_ref):
    cnt = cnt_ref[0:N, 0:1] + cnt_ref[N:2 * N, 0:1]
    dinv = lax.rsqrt(cnt + 1.0)
    h = jnp.dot(x_ref[...], w1_ref[...], preferred_element_type=jnp.float32)
    hs_ref[...] = h * dinv
    dinv_ref[...] = dinv


_tc_prep = pl.pallas_call(
    _tc_prep_body,
    out_shape=(jax.ShapeDtypeStruct((N, H), jnp.float32),
               jax.ShapeDtypeStruct((N, 1), jnp.float32)),
)


def _tc_mid_body(agg_ref, hs_ref, dinv_ref, b1_ref, w2_ref, hs2_ref):
    h = agg_ref[0:N, :] + agg_ref[N:2 * N, :] + hs_ref[...]
    h = jnp.maximum(dinv_ref[...] * h + b1_ref[...], 0.0)
    hs2_ref[...] = jnp.dot(h, w2_ref[...],
                           preferred_element_type=jnp.float32) * dinv_ref[...]


_tc_mid = pl.pallas_call(
    _tc_mid_body,
    out_shape=jax.ShapeDtypeStruct((N, H), jnp.float32),
)


def _tc_final_body(agg_ref, hs_ref, dinv_ref, b2_ref, batch_ref, wlin_ref,
                   blin_ref, out_ref):
    h = agg_ref[0:N, :] + agg_ref[N:2 * N, :] + hs_ref[...]
    h = jnp.maximum(dinv_ref[...] * h + b2_ref[...], 0.0)
    gids = lax.broadcasted_iota(jnp.int32, (G, N), 0)
    m = (batch_ref[...] == gids).astype(jnp.float32)
    sums = jnp.dot(m, h, preferred_element_type=jnp.float32)
    counts = jnp.sum(m, axis=1, keepdims=True)
    pooled = sums / jnp.maximum(counts, 1.0)
    logits = jnp.dot(pooled, wlin_ref[...],
                     preferred_element_type=jnp.float32) + blin_ref[...]
    z = logits - jnp.max(logits, axis=1, keepdims=True)
    out_ref[...] = z - jnp.log(jnp.sum(jnp.exp(z), axis=1, keepdims=True))


_tc_final = pl.pallas_call(
    _tc_final_body,
    out_shape=jax.ShapeDtypeStruct((G, C), jnp.float32),
)


def kernel(x, edge_index, batch, W1, b1, W2, b2, Wlin, blin):
    src = edge_index[0]
    dst = edge_index[1]
    zeros_nh = jnp.zeros((N, H), jnp.float32)
    zeros_n16 = jnp.zeros((N, 16), jnp.float32)
    ones_k16 = jnp.ones((K, 16), jnp.float32)
    cnt2 = _sc_count(dst, zeros_n16, ones_k16)
    hs1, dinv = _tc_prep(x, W1, cnt2)
    agg1 = _sc_agg(hs1, src, dst, zeros_nh)
    hs2 = _tc_mid(agg1, hs1, dinv, b1.reshape(1, H), W2)
    agg2 = _sc_agg(hs2, src, dst, zeros_nh)
    return _tc_final(agg2, hs2, dinv, b2.reshape(1, H), batch.reshape(1, N),
                     Wlin, blin.reshape(1, C))


# speedup vs baseline: 14.9430x; 14.9430x over previous
import functools

import jax
import jax.numpy as jnp
from jax import lax
from jax.experimental import pallas as pl
from jax.experimental.pallas import tpu as pltpu
from jax.experimental.pallas import tpu_sc as plsc

N = 10000
E = 320000
F_IN = 128
H = 64
C = 6
G = 64

NC = 2
NS = 16
NW = NC * NS
K = 80
EPW = E // NW
NCHUNK = EPW // K
NP = 10240
RPS = NP // NS

_mesh = plsc.VectorSubcoreMesh(core_axis_name="c", subcore_axis_name="s")
_sc_params = pltpu.CompilerParams(use_tc_tiling_on_sc=False)


@functools.partial(
    pl.kernel,
    out_type=jax.ShapeDtypeStruct((NC * NP, 16), jnp.float32),
    mesh=_mesh,
    scratch_types=[
        pltpu.VMEM((K,), jnp.int32),
        pltpu.VMEM((K, 16), jnp.float32),
        pltpu.VMEM_SHARED((NP, 16), jnp.float32),
    ],
    compiler_params=_sc_params,
)
def _sc_count(dst_hbm, zeros_hbm, ones_hbm, out_hbm, didx, ones_v, acc):
    cid = lax.axis_index("c")
    sid = lax.axis_index("s")
    wid = sid * NC + cid
    pltpu.sync_copy(zeros_hbm.at[pl.ds(sid * RPS, RPS)],
                    acc.at[pl.ds(sid * RPS, RPS)])
    pltpu.sync_copy(ones_hbm, ones_v)
    plsc.subcore_barrier()
    base = wid * EPW

    @pl.loop(0, NCHUNK)
    def _(c):
        pltpu.sync_copy(dst_hbm.at[pl.ds(base + c * K, K)], didx)
        pltpu.sync_copy(ones_v, acc.at[didx], add=True)

    plsc.subcore_barrier()
    pltpu.sync_copy(acc.at[pl.ds(sid * RPS, RPS)],
                    out_hbm.at[pl.ds(cid * NP + sid * RPS, RPS)])


@functools.partial(
    pl.kernel,
    out_type=jax.ShapeDtypeStruct((NC * NP, H), jnp.float32),
    mesh=_mesh,
    scratch_types=[
        pltpu.VMEM((K,), jnp.int32),
        pltpu.VMEM((K,), jnp.int32),
        pltpu.VMEM((K, H), jnp.float32),
        pltpu.VMEM_SHARED((NP, H), jnp.float32),
        pltpu.SemaphoreType.DMA,
    ],
    compiler_params=_sc_params,
)
def _sc_agg(hs_hbm, src_hbm, dst_hbm, zeros_hbm, out_hbm,
            sidx, didx, rows, acc, sem):
    cid = lax.axis_index("c")
    sid = lax.axis_index("s")
    wid = sid * NC + cid
    pltpu.sync_copy(zeros_hbm.at[pl.ds(sid * RPS, RPS)],
                    acc.at[pl.ds(sid * RPS, RPS)])
    plsc.subcore_barrier()
    base = wid * EPW

    @pl.loop(0, NCHUNK)
    def _(c):
        pltpu.sync_copy(src_hbm.at[pl.ds(base + c * K, K)], sidx)
        pltpu.sync_copy(dst_hbm.at[pl.ds(base + c * K, K)], didx)
        pltpu.async_copy(hs_hbm.at[sidx], rows, sem).wait()
        pltpu.sync_copy(rows, acc.at[didx], add=True)

    plsc.subcore_barrier()
    pltpu.sync_copy(acc.at[pl.ds(sid * RPS, RPS)],
                    out_hbm.at[pl.ds(cid * NP + sid * RPS, RPS)])


def _tc_prep_body(x_ref, w1_ref, cnt_ref, hs_ref, dinv_ref):
    cnt = cnt_ref[0:N, 0:1] + cnt_ref[NP:NP + N, 0:1]
    dinv = lax.rsqrt(cnt + 1.0)
    h = jnp.dot(x_ref[...], w1_ref[...], preferred_element_type=jnp.float32)
    hs_ref[...] = h * dinv
    dinv_ref[...] = dinv


_tc_prep = pl.pallas_call(
    _tc_prep_body,
    out_shape=(jax.ShapeDtypeStruct((N, H), jnp.float32),
               jax.ShapeDtypeStruct((N, 1), jnp.float32)),
)


def _tc_mid_body(agg_ref, hs_ref, dinv_ref, b1_ref, w2_ref, hs2_ref):
    h = agg_ref[0:N, :] + agg_ref[NP:NP + N, :] + hs_ref[...]
    h = jnp.maximum(dinv_ref[...] * h + b1_ref[...], 0.0)
    hs2_ref[...] = jnp.dot(h, w2_ref[...],
                           preferred_element_type=jnp.float32) * dinv_ref[...]


_tc_mid = pl.pallas_call(
    _tc_mid_body,
    out_shape=jax.ShapeDtypeStruct((N, H), jnp.float32),
)


def _tc_final_body(agg_ref, hs_ref, dinv_ref, b2_ref, batch_ref, wlin_ref,
                   blin_ref, out_ref):
    h = agg_ref[0:N, :] + agg_ref[NP:NP + N, :] + hs_ref[...]
    h = jnp.maximum(dinv_ref[...] * h + b2_ref[...], 0.0)
    gids = lax.broadcasted_iota(jnp.int32, (G, N), 0)
    m = (batch_ref[...] == gids).astype(jnp.float32)
    sums = jnp.dot(m, h, preferred_element_type=jnp.float32)
    counts = jnp.sum(m, axis=1, keepdims=True)
    pooled = sums / jnp.maximum(counts, 1.0)
    logits = jnp.dot(pooled, wlin_ref[...],
                     preferred_element_type=jnp.float32) + blin_ref[...]
    z = logits - jnp.max(logits, axis=1, keepdims=True)
    out_ref[...] = z - jnp.log(jnp.sum(jnp.exp(z), axis=1, keepdims=True))


_tc_final = pl.pallas_call(
    _tc_final_body,
    out_shape=jax.ShapeDtypeStruct((G, C), jnp.float32),
)


def kernel(x, edge_index, batch, W1, b1, W2, b2, Wlin, blin):
    src = edge_index[0]
    dst = edge_index[1]
    zeros_nh = jnp.zeros((NP, H), jnp.float32)
    zeros_n16 = jnp.zeros((NP, 16), jnp.float32)
    ones_k16 = jnp.ones((K, 16), jnp.float32)
    cnt2 = _sc_count(dst, zeros_n16, ones_k16)
    hs1, dinv = _tc_prep(x, W1, cnt2)
    agg1 = _sc_agg(hs1, src, dst, zeros_nh)
    hs2 = _tc_mid(agg1, hs1, dinv, b1.reshape(1, H), W2)
    agg2 = _sc_agg(hs2, src, dst, zeros_nh)
    return _tc_final(agg2, hs2, dinv, b2.reshape(1, H), batch.reshape(1, N),
                     Wlin, blin.reshape(1, C))


# speedup vs baseline: 29.3671x; 1.9653x over previous
import functools

import jax
import jax.numpy as jnp
from jax import lax
from jax.experimental import pallas as pl
from jax.experimental.pallas import tpu as pltpu
from jax.experimental.pallas import tpu_sc as plsc

N = 10000
E = 320000
F_IN = 128
H = 64
C = 6
G = 64

NC = 2
NS = 16
NW = NC * NS
K = 80
EPW = E // NW
NCHUNK = EPW // K
NP = 10240
RPS = NP // NS

_mesh = plsc.VectorSubcoreMesh(core_axis_name="c", subcore_axis_name="s")
_sc_params = pltpu.CompilerParams(use_tc_tiling_on_sc=False)


@functools.partial(
    pl.kernel,
    out_type=jax.ShapeDtypeStruct((NC * NP, 16), jnp.float32),
    mesh=_mesh,
    scratch_types=[
        pltpu.VMEM((NCHUNK, K), jnp.int32),
        pltpu.VMEM((K, 16), jnp.float32),
        pltpu.VMEM_SHARED((NP, 16), jnp.float32),
        pltpu.SemaphoreType.DMA,
        pltpu.SemaphoreType.DMA,
    ],
    compiler_params=_sc_params,
)
def _sc_count(dst_hbm, zeros_hbm, ones_hbm, out_hbm, didx, ones_v, acc,
              isem, ssem):
    cid = lax.axis_index("c")
    sid = lax.axis_index("s")
    wid = sid * NC + cid
    pltpu.async_copy(dst_hbm.at[wid], didx, isem)
    pltpu.sync_copy(ones_hbm, ones_v)
    pltpu.sync_copy(zeros_hbm.at[pl.ds(sid * RPS, RPS)],
                    acc.at[pl.ds(sid * RPS, RPS)])
    pltpu.make_async_copy(dst_hbm.at[wid], didx, isem).wait()
    plsc.subcore_barrier()

    @pl.loop(0, NCHUNK)
    def _(c):
        pltpu.async_copy(ones_v, acc.at[didx.at[c]], ssem, add=True)

    @pl.loop(0, NCHUNK)
    def _(c):
        pltpu.make_async_copy(ones_v, acc.at[didx.at[0]], ssem).wait()

    plsc.subcore_barrier()
    pltpu.sync_copy(acc.at[pl.ds(sid * RPS, RPS)],
                    out_hbm.at[pl.ds(cid * NP + sid * RPS, RPS)])


@functools.partial(
    pl.kernel,
    out_type=jax.ShapeDtypeStruct((NC * NP, H), jnp.float32),
    mesh=_mesh,
    scratch_types=[
        pltpu.VMEM((NCHUNK, K), jnp.int32),
        pltpu.VMEM((NCHUNK, K), jnp.int32),
        pltpu.VMEM((K, H), jnp.float32),
        pltpu.VMEM((K, H), jnp.float32),
        pltpu.VMEM_SHARED((NP, H), jnp.float32),
        pltpu.SemaphoreType.DMA,
        pltpu.SemaphoreType.DMA,
        pltpu.SemaphoreType.DMA,
        pltpu.SemaphoreType.DMA,
        pltpu.SemaphoreType.DMA,
    ],
    compiler_params=_sc_params,
)
def _sc_agg(hs_hbm, src_hbm, dst_hbm, zeros_hbm, out_hbm,
            sidx, didx, rows0, rows1, acc,
            gsem0, gsem1, ssem0, ssem1, isem):
    cid = lax.axis_index("c")
    sid = lax.axis_index("s")
    wid = sid * NC + cid
    pltpu.async_copy(src_hbm.at[wid], sidx, isem)
    pltpu.async_copy(dst_hbm.at[wid], didx, isem)
    pltpu.sync_copy(zeros_hbm.at[pl.ds(sid * RPS, RPS)],
                    acc.at[pl.ds(sid * RPS, RPS)])
    pltpu.make_async_copy(src_hbm.at[wid], sidx, isem).wait()
    pltpu.make_async_copy(dst_hbm.at[wid], didx, isem).wait()
    plsc.subcore_barrier()

    rows = (rows0, rows1)
    gsem = (gsem0, gsem1)
    ssem = (ssem0, ssem1)

    def start_gather(c, b):
        pltpu.async_copy(hs_hbm.at[sidx.at[c]], rows[b], gsem[b])

    def wait_gather(c, b):
        pltpu.make_async_copy(hs_hbm.at[sidx.at[c]], rows[b], gsem[b]).wait()

    def start_scatter(c, b):
        pltpu.async_copy(rows[b], acc.at[didx.at[c]], ssem[b], add=True)

    def wait_scatter(c, b):
        pltpu.make_async_copy(rows[b], acc.at[didx.at[c]], ssem[b]).wait()

    start_gather(0, 0)
    wait_gather(0, 0)
    start_scatter(0, 0)
    start_gather(1, 1)

    @pl.loop(1, NCHUNK - 2, step=2)
    def _(c0):
        for j in range(2):
            c = c0 + j
            b = (1 + j) % 2
            wait_gather(c, b)
            start_scatter(c, b)
            wait_scatter(c - 1, 1 - b)
            start_gather(c + 1, 1 - b)

    wait_gather(NCHUNK - 2, 1)
    start_scatter(NCHUNK - 2, 1)
    wait_scatter(NCHUNK - 3, 0)
    start_gather(NCHUNK - 1, 0)
    wait_gather(NCHUNK - 1, 0)
    start_scatter(NCHUNK - 1, 0)
    wait_scatter(NCHUNK - 2, 1)
    wait_scatter(NCHUNK - 1, 0)

    plsc.subcore_barrier()
    pltpu.sync_copy(acc.at[pl.ds(sid * RPS, RPS)],
                    out_hbm.at[pl.ds(cid * NP + sid * RPS, RPS)])


def _tc_prep_body(x_ref, w1_ref, cnt_ref, hs_ref, dinv_ref):
    cnt = cnt_ref[0:N, 0:1] + cnt_ref[NP:NP + N, 0:1]
    dinv = lax.rsqrt(cnt + 1.0)
    h = jnp.dot(x_ref[...], w1_ref[...], preferred_element_type=jnp.float32)
    hs_ref[...] = h * dinv
    dinv_ref[...] = dinv


_tc_prep = pl.pallas_call(
    _tc_prep_body,
    out_shape=(jax.ShapeDtypeStruct((N, H), jnp.float32),
               jax.ShapeDtypeStruct((N, 1), jnp.float32)),
)


def _tc_mid_body(agg_ref, hs_ref, dinv_ref, b1_ref, w2_ref, hs2_ref):
    h = agg_ref[0:N, :] + agg_ref[NP:NP + N, :] + hs_ref[...]
    h = jnp.maximum(dinv_ref[...] * h + b1_ref[...], 0.0)
    hs2_ref[...] = jnp.dot(h, w2_ref[...],
                           preferred_element_type=jnp.float32) * dinv_ref[...]


_tc_mid = pl.pallas_call(
    _tc_mid_body,
    out_shape=jax.ShapeDtypeStruct((N, H), jnp.float32),
)


def _tc_final_body(agg_ref, hs_ref, dinv_ref, b2_ref, batch_ref, wlin_ref,
                   blin_ref, out_ref):
    h = agg_ref[0:N, :] + agg_ref[NP:NP + N, :] + hs_ref[...]
    h = jnp.maximum(dinv_ref[...] * h + b2_ref[...], 0.0)
    gids = lax.broadcasted_iota(jnp.int32, (G, N), 0)
    m = (batch_ref[...] == gids).astype(jnp.float32)
    sums = jnp.dot(m, h, preferred_element_type=jnp.float32)
    counts = jnp.sum(m, axis=1, keepdims=True)
    pooled = sums / jnp.maximum(counts, 1.0)
    logits = jnp.dot(pooled, wlin_ref[...],
                     preferred_element_type=jnp.float32) + blin_ref[...]
    z = logits - jnp.max(logits, axis=1, keepdims=True)
    out_ref[...] = z - jnp.log(jnp.sum(jnp.exp(z), axis=1, keepdims=True))


_tc_final = pl.pallas_call(
    _tc_final_body,
    out_shape=jax.ShapeDtypeStruct((G, C), jnp.float32),
)


def kernel(x, edge_index, batch, W1, b1, W2, b2, Wlin, blin):
    src = edge_index[0].reshape(NW, NCHUNK, K)
    dst = edge_index[1].reshape(NW, NCHUNK, K)
    zeros_nh = jnp.zeros((NP, H), jnp.float32)
    zeros_n16 = jnp.zeros((NP, 16), jnp.float32)
    ones_k16 = jnp.ones((K, 16), jnp.float32)
    cnt2 = _sc_count(dst, zeros_n16, ones_k16)
    hs1, dinv = _tc_prep(x, W1, cnt2)
    agg1 = _sc_agg(hs1, src, dst, zeros_nh)
    hs2 = _tc_mid(agg1, hs1, dinv, b1.reshape(1, H), W2)
    agg2 = _sc_agg(hs2, src, dst, zeros_nh)
    return _tc_final(agg2, hs2, dinv, b2.reshape(1, H), batch.reshape(1, N),
                     Wlin, blin.reshape(1, C))


# speedup vs baseline: 44.2247x; 1.5059x over previous
import functools

import jax
import jax.numpy as jnp
from jax import lax
from jax.experimental import pallas as pl
from jax.experimental.pallas import tpu as pltpu
from jax.experimental.pallas import tpu_sc as plsc

N = 10000
E = 320000
F_IN = 128
H = 64
C = 6
G = 64

NC = 2
NS = 16
NW = NC * NS
K = 80
EPW = E // NW
NCHUNK = EPW // K
NP = 10240
RPS = NP // NS

_mesh = plsc.VectorSubcoreMesh(core_axis_name="c", subcore_axis_name="s")
_sc_params = pltpu.CompilerParams(use_tc_tiling_on_sc=False)


@functools.partial(
    pl.kernel,
    out_type=jax.ShapeDtypeStruct((NC * NP, 16), jnp.float32),
    mesh=_mesh,
    scratch_types=[
        pltpu.VMEM((NCHUNK, K), jnp.int32),
        pltpu.VMEM((K, 16), jnp.float32),
        pltpu.VMEM_SHARED((NP, 16), jnp.float32),
        pltpu.SemaphoreType.DMA,
        pltpu.SemaphoreType.DMA,
    ],
    compiler_params=_sc_params,
)
def _sc_count(dst_hbm, zeros_hbm, ones_hbm, out_hbm, didx, ones_v, acc,
              isem, ssem):
    cid = lax.axis_index("c")
    sid = lax.axis_index("s")
    wid = sid * NC + cid
    pltpu.async_copy(dst_hbm.at[wid], didx, isem)
    pltpu.sync_copy(ones_hbm, ones_v)
    pltpu.sync_copy(zeros_hbm.at[pl.ds(sid * RPS, RPS)],
                    acc.at[pl.ds(sid * RPS, RPS)])
    pltpu.make_async_copy(dst_hbm.at[wid], didx, isem).wait()
    plsc.subcore_barrier()

    @pl.loop(0, NCHUNK)
    def _(c):
        pltpu.async_copy(ones_v, acc.at[didx.at[c]], ssem, add=True)

    @pl.loop(0, NCHUNK)
    def _(c):
        pltpu.make_async_copy(ones_v, acc.at[didx.at[0]], ssem).wait()

    plsc.subcore_barrier()
    pltpu.sync_copy(acc.at[pl.ds(sid * RPS, RPS)],
                    out_hbm.at[pl.ds(cid * NP + sid * RPS, RPS)])


@functools.partial(
    pl.kernel,
    out_type=jax.ShapeDtypeStruct((NC * NP, H), jnp.float32),
    mesh=_mesh,
    scratch_types=[
        pltpu.VMEM((NCHUNK, K), jnp.int32),
        pltpu.VMEM((NCHUNK, K), jnp.int32),
        pltpu.VMEM((K, H), jnp.float32),
        pltpu.VMEM((K, H), jnp.float32),
        pltpu.VMEM((K, H), jnp.float32),
        pltpu.VMEM((K, H), jnp.float32),
        pltpu.VMEM_SHARED((NP, H), jnp.float32),
        pltpu.SemaphoreType.DMA,
        pltpu.SemaphoreType.DMA,
        pltpu.SemaphoreType.DMA,
        pltpu.SemaphoreType.DMA,
        pltpu.SemaphoreType.DMA,
        pltpu.SemaphoreType.DMA,
        pltpu.SemaphoreType.DMA,
        pltpu.SemaphoreType.DMA,
        pltpu.SemaphoreType.DMA,
    ],
    compiler_params=_sc_params,
)
def _sc_agg(hs_hbm, src_hbm, dst_hbm, zeros_hbm, out_hbm,
            sidx, didx, rows0, rows1, rows2, rows3, acc,
            gsem0, gsem1, gsem2, gsem3, ssem0, ssem1, ssem2, ssem3, isem):
    cid = lax.axis_index("c")
    sid = lax.axis_index("s")
    wid = sid * NC + cid
    pltpu.async_copy(src_hbm.at[wid], sidx, isem)
    pltpu.async_copy(dst_hbm.at[wid], didx, isem)
    pltpu.sync_copy(zeros_hbm.at[pl.ds(sid * RPS, RPS)],
                    acc.at[pl.ds(sid * RPS, RPS)])
    pltpu.make_async_copy(src_hbm.at[wid], sidx, isem).wait()
    pltpu.make_async_copy(dst_hbm.at[wid], didx, isem).wait()
    plsc.subcore_barrier()

    rows = (rows0, rows1, rows2, rows3)
    gsem = (gsem0, gsem1, gsem2, gsem3)
    ssem = (ssem0, ssem1, ssem2, ssem3)

    def start_gather(c, b):
        pltpu.async_copy(hs_hbm.at[sidx.at[c]], rows[b], gsem[b])

    def wait_gather(c, b):
        pltpu.make_async_copy(hs_hbm.at[sidx.at[c]], rows[b], gsem[b]).wait()

    def start_scatter(c, b):
        pltpu.async_copy(rows[b], acc.at[didx.at[c]], ssem[b], add=True)

    def wait_scatter(c, b):
        pltpu.make_async_copy(rows[b], acc.at[didx.at[c]], ssem[b]).wait()

    start_gather(0, 0)
    start_gather(1, 1)
    start_gather(2, 2)
    wait_gather(0, 0)
    start_scatter(0, 0)
    start_gather(3, 3)

    @pl.loop(1, NCHUNK - 4, step=4)
    def _(c0):
        for j in range(4):
            c = c0 + j
            b = (1 + j) % 4
            wait_gather(c, b)
            start_scatter(c, b)
            wait_scatter(c - 1, (b + 3) % 4)
            start_gather(c + 3, (b + 3) % 4)

    c = NCHUNK - 4
    wait_gather(c, c % 4)
    start_scatter(c, c % 4)
    wait_scatter(c - 1, (c + 3) % 4)
    start_gather(c + 3, (c + 3) % 4)
    for c in range(NCHUNK - 3, NCHUNK):
        wait_gather(c, c % 4)
        start_scatter(c, c % 4)
        wait_scatter(c - 1, (c - 1) % 4)
    wait_scatter(NCHUNK - 1, (NCHUNK - 1) % 4)

    plsc.subcore_barrier()
    pltpu.sync_copy(acc.at[pl.ds(sid * RPS, RPS)],
                    out_hbm.at[pl.ds(cid * NP + sid * RPS, RPS)])


def _tc_prep_body(x_ref, w1_ref, cnt_ref, hs_ref, dinv_ref):
    cnt = cnt_ref[0:N, 0:1] + cnt_ref[NP:NP + N, 0:1]
    dinv = lax.rsqrt(cnt + 1.0)
    h = jnp.dot(x_ref[...], w1_ref[...], preferred_element_type=jnp.float32)
    hs_ref[...] = h * dinv
    dinv_ref[...] = dinv


_tc_prep = pl.pallas_call(
    _tc_prep_body,
    out_shape=(jax.ShapeDtypeStruct((N, H), jnp.float32),
               jax.ShapeDtypeStruct((N, 1), jnp.float32)),
)


def _tc_mid_body(agg_ref, hs_ref, dinv_ref, b1_ref, w2_ref, hs2_ref):
    h = agg_ref[0:N, :] + agg_ref[NP:NP + N, :] + hs_ref[...]
    h = jnp.maximum(dinv_ref[...] * h + b1_ref[...], 0.0)
    hs2_ref[...] = jnp.dot(h, w2_ref[...],
                           preferred_element_type=jnp.float32) * dinv_ref[...]


_tc_mid = pl.pallas_call(
    _tc_mid_body,
    out_shape=jax.ShapeDtypeStruct((N, H), jnp.float32),
)


def _tc_final_body(agg_ref, hs_ref, dinv_ref, b2_ref, batch_ref, wlin_ref,
                   blin_ref, out_ref):
    h = agg_ref[0:N, :] + agg_ref[NP:NP + N, :] + hs_ref[...]
    h = jnp.maximum(dinv_ref[...] * h + b2_ref[...], 0.0)
    gids = lax.broadcasted_iota(jnp.int32, (G, N), 0)
    m = (batch_ref[...] == gids).astype(jnp.float32)
    sums = jnp.dot(m, h, preferred_element_type=jnp.float32)
    counts = jnp.sum(m, axis=1, keepdims=True)
    pooled = sums / jnp.maximum(counts, 1.0)
    logits = jnp.dot(pooled, wlin_ref[...],
                     preferred_element_type=jnp.float32) + blin_ref[...]
    z = logits - jnp.max(logits, axis=1, keepdims=True)
    out_ref[...] = z - jnp.log(jnp.sum(jnp.exp(z), axis=1, keepdims=True))


_tc_final = pl.pallas_call(
    _tc_final_body,
    out_shape=jax.ShapeDtypeStruct((G, C), jnp.float32),
)


def kernel(x, edge_index, batch, W1, b1, W2, b2, Wlin, blin):
    src = edge_index[0].reshape(NW, NCHUNK, K)
    dst = edge_index[1].reshape(NW, NCHUNK, K)
    zeros_nh = jnp.zeros((NP, H), jnp.float32)
    zeros_n16 = jnp.zeros((NP, 16), jnp.float32)
    ones_k16 = jnp.ones((K, 16), jnp.float32)
    cnt2 = _sc_count(dst, zeros_n16, ones_k16)
    hs1, dinv = _tc_prep(x, W1, cnt2)
    agg1 = _sc_agg(hs1, src, dst, zeros_nh)
    hs2 = _tc_mid(agg1, hs1, dinv, b1.reshape(1, H), W2)
    agg2 = _sc_agg(hs2, src, dst, zeros_nh)
    return _tc_final(agg2, hs2, dinv, b2.reshape(1, H), batch.reshape(1, N),
                     Wlin, blin.reshape(1, C))


# speedup vs baseline: 46.5685x; 1.0530x over previous
import functools

import jax
import jax.numpy as jnp
from jax import lax
from jax.experimental import pallas as pl
from jax.experimental.pallas import tpu as pltpu
from jax.experimental.pallas import tpu_sc as plsc

N = 10000
E = 320000
F_IN = 128
H = 64
C = 6
G = 64

NC = 2
NS = 16
NW = NC * NS
K = 80
EPW = E // NW
NCHUNK = EPW // K
NP = 10240
RPS = NP // NS

_mesh = plsc.VectorSubcoreMesh(core_axis_name="c", subcore_axis_name="s")
_sc_params = pltpu.CompilerParams(use_tc_tiling_on_sc=False)


@functools.partial(
    pl.kernel,
    out_type=jax.ShapeDtypeStruct((NC * NP, 16), jnp.float32),
    mesh=_mesh,
    scratch_types=[
        pltpu.VMEM((NCHUNK, K), jnp.int32),
        pltpu.VMEM((K, 16), jnp.float32),
        pltpu.VMEM_SHARED((NP, 16), jnp.float32),
        pltpu.SemaphoreType.DMA,
        pltpu.SemaphoreType.DMA,
    ],
    compiler_params=_sc_params,
)
def _sc_count(eidx_hbm, zeros_hbm, ones_hbm, out_hbm, didx, ones_v, acc,
              isem, ssem):
    cid = lax.axis_index("c")
    sid = lax.axis_index("s")
    wid = sid * NC + cid
    pltpu.async_copy(eidx_hbm.at[1].at[wid], didx, isem)
    pltpu.sync_copy(ones_hbm, ones_v)
    pltpu.sync_copy(zeros_hbm.at[pl.ds(sid * RPS, RPS)],
                    acc.at[pl.ds(sid * RPS, RPS)])
    pltpu.make_async_copy(eidx_hbm.at[1].at[wid], didx, isem).wait()
    plsc.subcore_barrier()

    @pl.loop(0, NCHUNK)
    def _(c):
        pltpu.async_copy(ones_v, acc.at[didx.at[c]], ssem, add=True)

    @pl.loop(0, NCHUNK)
    def _(c):
        pltpu.make_async_copy(ones_v, acc.at[didx.at[0]], ssem).wait()

    plsc.subcore_barrier()
    pltpu.sync_copy(acc.at[pl.ds(sid * RPS, RPS)],
                    out_hbm.at[pl.ds(cid * NP + sid * RPS, RPS)])


@functools.partial(
    pl.kernel,
    out_type=jax.ShapeDtypeStruct((NC * NP, H), jnp.float32),
    mesh=_mesh,
    scratch_types=[
        pltpu.VMEM((NCHUNK, K), jnp.int32),
        pltpu.VMEM((NCHUNK, K), jnp.int32),
        pltpu.VMEM((K, H), jnp.float32),
        pltpu.VMEM((K, H), jnp.float32),
        pltpu.VMEM((K, H), jnp.float32),
        pltpu.VMEM((K, H), jnp.float32),
        pltpu.VMEM_SHARED((NP, H), jnp.float32),
        pltpu.SemaphoreType.DMA,
        pltpu.SemaphoreType.DMA,
        pltpu.SemaphoreType.DMA,
        pltpu.SemaphoreType.DMA,
        pltpu.SemaphoreType.DMA,
        pltpu.SemaphoreType.DMA,
        pltpu.SemaphoreType.DMA,
        pltpu.SemaphoreType.DMA,
        pltpu.SemaphoreType.DMA,
    ],
    compiler_params=_sc_params,
)
def _sc_agg(hs_hbm, eidx_hbm, zeros_hbm, out_hbm,
            sidx, didx, rows0, rows1, rows2, rows3, acc,
            gsem0, gsem1, gsem2, gsem3, ssem0, ssem1, ssem2, ssem3, isem):
    cid = lax.axis_index("c")
    sid = lax.axis_index("s")
    wid = sid * NC + cid
    pltpu.async_copy(eidx_hbm.at[0].at[wid], sidx, isem)
    pltpu.async_copy(eidx_hbm.at[1].at[wid], didx, isem)
    pltpu.sync_copy(zeros_hbm.at[pl.ds(sid * RPS, RPS)],
                    acc.at[pl.ds(sid * RPS, RPS)])
    pltpu.make_async_copy(eidx_hbm.at[0].at[wid], sidx, isem).wait()
    pltpu.make_async_copy(eidx_hbm.at[1].at[wid], didx, isem).wait()
    plsc.subcore_barrier()

    rows = (rows0, rows1, rows2, rows3)
    gsem = (gsem0, gsem1, gsem2, gsem3)
    ssem = (ssem0, ssem1, ssem2, ssem3)

    def start_gather(c, b):
        pltpu.async_copy(hs_hbm.at[sidx.at[c]], rows[b], gsem[b])

    def wait_gather(c, b):
        pltpu.make_async_copy(hs_hbm.at[sidx.at[c]], rows[b], gsem[b]).wait()

    def start_scatter(c, b):
        pltpu.async_copy(rows[b], acc.at[didx.at[c]], ssem[b], add=True)

    def wait_scatter(c, b):
        pltpu.make_async_copy(rows[b], acc.at[didx.at[c]], ssem[b]).wait()

    start_gather(0, 0)
    start_gather(1, 1)
    start_gather(2, 2)
    wait_gather(0, 0)
    start_scatter(0, 0)
    start_gather(3, 3)

    @pl.loop(1, NCHUNK - 4, step=4)
    def _(c0):
        for j in range(4):
            c = c0 + j
            b = (1 + j) % 4
            wait_gather(c, b)
            start_scatter(c, b)
            wait_scatter(c - 1, (b + 3) % 4)
            start_gather(c + 3, (b + 3) % 4)

    c = NCHUNK - 4
    wait_gather(c, c % 4)
    start_scatter(c, c % 4)
    wait_scatter(c - 1, (c + 3) % 4)
    start_gather(c + 3, (c + 3) % 4)
    for c in range(NCHUNK - 3, NCHUNK):
        wait_gather(c, c % 4)
        start_scatter(c, c % 4)
        wait_scatter(c - 1, (c - 1) % 4)
    wait_scatter(NCHUNK - 1, (NCHUNK - 1) % 4)

    plsc.subcore_barrier()
    pltpu.sync_copy(acc.at[pl.ds(sid * RPS, RPS)],
                    out_hbm.at[pl.ds(cid * NP + sid * RPS, RPS)])


def _tc_mm1_body(x_ref, w1_ref, h_ref):
    h_ref[...] = jnp.dot(x_ref[...], w1_ref[...],
                         preferred_element_type=jnp.float32)


_tc_mm1 = pl.pallas_call(
    _tc_mm1_body,
    out_shape=jax.ShapeDtypeStruct((N, H), jnp.float32),
)


def _tc_scale_body(h_ref, cnt_ref, hs_ref, dinv_ref):
    cnt = cnt_ref[0:N, 0:1] + cnt_ref[NP:NP + N, 0:1]
    dinv = lax.rsqrt(cnt + 1.0)
    hs_ref[...] = h_ref[...] * dinv
    dinv_ref[...] = dinv


_tc_scale = pl.pallas_call(
    _tc_scale_body,
    out_shape=(jax.ShapeDtypeStruct((N, H), jnp.float32),
               jax.ShapeDtypeStruct((N, 1), jnp.float32)),
)


def _tc_mid_body(agg_ref, hs_ref, dinv_ref, b1_ref, w2_ref, hs2_ref):
    h = agg_ref[0:N, :] + agg_ref[NP:NP + N, :] + hs_ref[...]
    h = jnp.maximum(dinv_ref[...] * h + b1_ref[...], 0.0)
    hs2_ref[...] = jnp.dot(h, w2_ref[...],
                           preferred_element_type=jnp.float32) * dinv_ref[...]


_tc_mid = pl.pallas_call(
    _tc_mid_body,
    out_shape=jax.ShapeDtypeStruct((N, H), jnp.float32),
)


def _tc_final_body(agg_ref, hs_ref, dinv_ref, b2_ref, batch_ref, wlin_ref,
                   blin_ref, out_ref):
    h = agg_ref[0:N, :] + agg_ref[NP:NP + N, :] + hs_ref[...]
    h = jnp.maximum(dinv_ref[...] * h + b2_ref[...], 0.0)
    gids = lax.broadcasted_iota(jnp.int32, (G, N), 0)
    m = (batch_ref[...] == gids).astype(jnp.float32)
    sums = jnp.dot(m, h, preferred_element_type=jnp.float32)
    counts = jnp.sum(m, axis=1, keepdims=True)
    pooled = sums / jnp.maximum(counts, 1.0)
    logits = jnp.dot(pooled, wlin_ref[...],
                     preferred_element_type=jnp.float32) + blin_ref[...]
    z = logits - jnp.max(logits, axis=1, keepdims=True)
    out_ref[...] = z - jnp.log(jnp.sum(jnp.exp(z), axis=1, keepdims=True))


_tc_final = pl.pallas_call(
    _tc_final_body,
    out_shape=jax.ShapeDtypeStruct((G, C), jnp.float32),
)


def kernel(x, edge_index, batch, W1, b1, W2, b2, Wlin, blin):
    eidx = edge_index.reshape(2, NW, NCHUNK, K)
    zeros_nh = jnp.zeros((NP, H), jnp.float32)
    zeros_n16 = jnp.zeros((NP, 16), jnp.float32)
    ones_k16 = jnp.ones((K, 16), jnp.float32)
    cnt2 = _sc_count(eidx, zeros_n16, ones_k16)
    h1 = _tc_mm1(x, W1)
    hs1, dinv = _tc_scale(h1, cnt2)
    agg1 = _sc_agg(hs1, eidx, zeros_nh)
    hs2 = _tc_mid(agg1, hs1, dinv, b1.reshape(1, H), W2)
    agg2 = _sc_agg(hs2, eidx, zeros_nh)
    return _tc_final(agg2, hs2, dinv, b2.reshape(1, H), batch.reshape(1, N),
                     Wlin, blin.reshape(1, C))


# speedup vs baseline: 52.2932x; 1.1229x over previous
import functools

import jax
import jax.numpy as jnp
from jax import lax
from jax.experimental import pallas as pl
from jax.experimental.pallas import tpu as pltpu
from jax.experimental.pallas import tpu_sc as plsc

N = 10000
E = 320000
F_IN = 128
H = 64
C = 6
G = 64

NC = 2
NS = 16
NW = NC * NS
K = 80
EPW = E // NW
NCHUNK = EPW // K
NP = 10240
RPS = NP // NS

_mesh = plsc.VectorSubcoreMesh(core_axis_name="c", subcore_axis_name="s")
_sc_params = pltpu.CompilerParams(use_tc_tiling_on_sc=False)


@functools.partial(
    pl.kernel,
    out_type=jax.ShapeDtypeStruct((NC * NP, H), jnp.float32),
    mesh=_mesh,
    scratch_types=[
        pltpu.VMEM((NCHUNK, K), jnp.int32),
        pltpu.VMEM((K, H), jnp.float32),
        pltpu.VMEM_SHARED((NP, H), jnp.float32),
        pltpu.SemaphoreType.DMA,
        pltpu.SemaphoreType.DMA,
    ],
    compiler_params=_sc_params,
)
def _sc_count(eidx_hbm, zeros_hbm, ones_hbm, out_hbm, didx, ones_v, acc,
              isem, ssem):
    cid = lax.axis_index("c")
    sid = lax.axis_index("s")
    wid = sid * NC + cid
    pltpu.async_copy(eidx_hbm.at[1].at[wid], didx, isem)
    pltpu.sync_copy(ones_hbm, ones_v)
    pltpu.sync_copy(zeros_hbm.at[pl.ds(sid * RPS, RPS)],
                    acc.at[pl.ds(sid * RPS, RPS)])
    pltpu.make_async_copy(eidx_hbm.at[1].at[wid], didx, isem).wait()
    plsc.subcore_barrier()

    @pl.loop(0, NCHUNK)
    def _(c):
        pltpu.async_copy(ones_v, acc.at[didx.at[c]], ssem, add=True)

    @pl.loop(0, NCHUNK)
    def _(c):
        pltpu.make_async_copy(ones_v, acc.at[didx.at[0]], ssem).wait()

    plsc.subcore_barrier()
    pltpu.sync_copy(acc.at[pl.ds(sid * RPS, RPS)],
                    out_hbm.at[pl.ds(cid * NP + sid * RPS, RPS)])


@functools.partial(
    pl.kernel,
    out_type=jax.ShapeDtypeStruct((NC * NP, H), jnp.float32),
    mesh=_mesh,
    scratch_types=[
        pltpu.VMEM((NCHUNK, K), jnp.int32),
        pltpu.VMEM((NCHUNK, K), jnp.int32),
        pltpu.VMEM((K, H), jnp.float32),
        pltpu.VMEM((K, H), jnp.float32),
        pltpu.VMEM((K, H), jnp.float32),
        pltpu.VMEM((K, H), jnp.float32),
        pltpu.VMEM_SHARED((NP, H), jnp.float32),
        pltpu.SemaphoreType.DMA,
        pltpu.SemaphoreType.DMA,
        pltpu.SemaphoreType.DMA,
        pltpu.SemaphoreType.DMA,
        pltpu.SemaphoreType.DMA,
        pltpu.SemaphoreType.DMA,
        pltpu.SemaphoreType.DMA,
        pltpu.SemaphoreType.DMA,
        pltpu.SemaphoreType.DMA,
    ],
    compiler_params=_sc_params,
)
def _sc_agg(hs_hbm, eidx_hbm, zeros_hbm, out_hbm,
            sidx, didx, rows0, rows1, rows2, rows3, acc,
            gsem0, gsem1, gsem2, gsem3, ssem0, ssem1, ssem2, ssem3, isem):
    cid = lax.axis_index("c")
    sid = lax.axis_index("s")
    wid = sid * NC + cid
    pltpu.async_copy(eidx_hbm.at[0].at[wid], sidx, isem)
    pltpu.async_copy(eidx_hbm.at[1].at[wid], didx, isem)
    pltpu.sync_copy(zeros_hbm.at[pl.ds(sid * RPS, RPS)],
                    acc.at[pl.ds(sid * RPS, RPS)])
    pltpu.make_async_copy(eidx_hbm.at[0].at[wid], sidx, isem).wait()
    pltpu.make_async_copy(eidx_hbm.at[1].at[wid], didx, isem).wait()
    plsc.subcore_barrier()

    rows = (rows0, rows1, rows2, rows3)
    gsem = (gsem0, gsem1, gsem2, gsem3)
    ssem = (ssem0, ssem1, ssem2, ssem3)

    def start_gather(c, b):
        pltpu.async_copy(hs_hbm.at[sidx.at[c]], rows[b], gsem[b])

    def wait_gather(c, b):
        pltpu.make_async_copy(hs_hbm.at[sidx.at[c]], rows[b], gsem[b]).wait()

    def start_scatter(c, b):
        pltpu.async_copy(rows[b], acc.at[didx.at[c]], ssem[b], add=True)

    def wait_scatter(c, b):
        pltpu.make_async_copy(rows[b], acc.at[didx.at[c]], ssem[b]).wait()

    start_gather(0, 0)
    start_gather(1, 1)
    start_gather(2, 2)
    wait_gather(0, 0)
    start_scatter(0, 0)
    start_gather(3, 3)

    @pl.loop(1, NCHUNK - 4, step=4)
    def _(c0):
        for j in range(4):
            c = c0 + j
            b = (1 + j) % 4
            wait_gather(c, b)
            start_scatter(c, b)
            wait_scatter(c - 1, (b + 3) % 4)
            start_gather(c + 3, (b + 3) % 4)

    c = NCHUNK - 4
    wait_gather(c, c % 4)
    start_scatter(c, c % 4)
    wait_scatter(c - 1, (c + 3) % 4)
    start_gather(c + 3, (c + 3) % 4)
    for c in range(NCHUNK - 3, NCHUNK):
        wait_gather(c, c % 4)
        start_scatter(c, c % 4)
        wait_scatter(c - 1, (c - 1) % 4)
    wait_scatter(NCHUNK - 1, (NCHUNK - 1) % 4)

    plsc.subcore_barrier()
    pltpu.sync_copy(acc.at[pl.ds(sid * RPS, RPS)],
                    out_hbm.at[pl.ds(cid * NP + sid * RPS, RPS)])


NPAIR = N // 2
NPP = NP // 2


def _tc_mm1_body(x_ref, w1b_ref, h_ref):
    h_ref[...] = jnp.dot(x_ref[...], w1b_ref[...],
                         preferred_element_type=jnp.float32)


_tc_mm1 = pl.pallas_call(
    _tc_mm1_body,
    out_shape=jax.ShapeDtypeStruct((NPAIR, 2 * H), jnp.float32),
)


def _tc_scale_body(h_ref, cnt_ref, hs_ref, dinv_ref):
    cnt = cnt_ref[0:NPAIR, :] + cnt_ref[NPP:NPP + NPAIR, :]
    dinv = lax.rsqrt(cnt + 1.0)
    hs_ref[...] = h_ref[...] * dinv
    dinv_ref[...] = dinv


_tc_scale = pl.pallas_call(
    _tc_scale_body,
    out_shape=(jax.ShapeDtypeStruct((NPAIR, 2 * H), jnp.float32),
               jax.ShapeDtypeStruct((NPAIR, 2 * H), jnp.float32)),
)


def _tc_mid_body(agg_ref, hs_ref, dinv_ref, b1_ref, w2b_ref, hs2_ref):
    h = agg_ref[0:NPAIR, :] + agg_ref[NPP:NPP + NPAIR, :] + hs_ref[...]
    h = jnp.maximum(dinv_ref[...] * h + b1_ref[...], 0.0)
    hs2_ref[...] = jnp.dot(h, w2b_ref[...],
                           preferred_element_type=jnp.float32) * dinv_ref[...]


_tc_mid = pl.pallas_call(
    _tc_mid_body,
    out_shape=jax.ShapeDtypeStruct((NPAIR, 2 * H), jnp.float32),
)


def _tc_final_body(agg_ref, hs_ref, dinv_ref, b2_ref, be_ref, bo_ref,
                   wlin_ref, blin_ref, out_ref):
    h = agg_ref[0:NPAIR, :] + agg_ref[NPP:NPP + NPAIR, :] + hs_ref[...]
    h = jnp.maximum(dinv_ref[...] * h + b2_ref[...], 0.0)
    gids = lax.broadcasted_iota(jnp.int32, (G, NPAIR), 0)
    me = (be_ref[...] == gids).astype(jnp.float32)
    mo = (bo_ref[...] == gids).astype(jnp.float32)
    sums = (jnp.dot(me, h[:, 0:H], preferred_element_type=jnp.float32)
            + jnp.dot(mo, h[:, H:2 * H], preferred_element_type=jnp.float32))
    counts = jnp.sum(me + mo, axis=1, keepdims=True)
    pooled = sums / jnp.maximum(counts, 1.0)
    logits = jnp.dot(pooled, wlin_ref[...],
                     preferred_element_type=jnp.float32) + blin_ref[...]
    z = logits - jnp.max(logits, axis=1, keepdims=True)
    out_ref[...] = z - jnp.log(jnp.sum(jnp.exp(z), axis=1, keepdims=True))


_tc_final = pl.pallas_call(
    _tc_final_body,
    out_shape=jax.ShapeDtypeStruct((G, C), jnp.float32),
)


def _blockdiag(w, k):
    z = jnp.zeros((k, w.shape[0], k, w.shape[1]), jnp.float32)
    z = z.at[jnp.arange(k), :, jnp.arange(k), :].set(w)
    return z.reshape(k * w.shape[0], k * w.shape[1])


def kernel(x, edge_index, batch, W1, b1, W2, b2, Wlin, blin):
    eidx = edge_index.reshape(2, NW, NCHUNK, K)
    zeros_nh = jnp.zeros((NP, H), jnp.float32)
    ones_kh = jnp.ones((K, H), jnp.float32)
    w1b = _blockdiag(W1, 2)
    w2b = _blockdiag(W2, 2)
    b1p = jnp.concatenate([b1, b1]).reshape(1, 2 * H)
    b2p = jnp.concatenate([b2, b2]).reshape(1, 2 * H)
    be = batch[0::2].reshape(1, NPAIR)
    bo = batch[1::2].reshape(1, NPAIR)
    cnt2 = _sc_count(eidx, zeros_nh, ones_kh)
    h1p = _tc_mm1(x.reshape(NPAIR, 2 * F_IN), w1b)
    hs1p, dinv2p = _tc_scale(h1p, cnt2.reshape(NP, 2 * H))
    agg1 = _sc_agg(hs1p.reshape(N, H), eidx, zeros_nh)
    hs2p = _tc_mid(agg1.reshape(NP, 2 * H), hs1p, dinv2p, b1p, w2b)
    agg2 = _sc_agg(hs2p.reshape(N, H), eidx, zeros_nh)
    return _tc_final(agg2.reshape(NP, 2 * H), hs2p, dinv2p, b2p, be, bo,
                     Wlin, blin.reshape(1, C))


# speedup vs baseline: 58.1416x; 1.1118x over previous
import functools

import jax
import jax.numpy as jnp
from jax import lax
from jax.experimental import pallas as pl
from jax.experimental.pallas import tpu as pltpu
from jax.experimental.pallas import tpu_sc as plsc

N = 10000
E = 320000
F_IN = 128
H = 64
C = 6
G = 64

NC = 2
NS = 16
NW = NC * NS
K = 80
EPW = E // NW
NCHUNK = EPW // K
NP = 10240
RPS = NP // NS

_mesh = plsc.VectorSubcoreMesh(core_axis_name="c", subcore_axis_name="s")
_sc_params = pltpu.CompilerParams(use_tc_tiling_on_sc=False)


@functools.partial(
    pl.kernel,
    out_type=jax.ShapeDtypeStruct((NC * NP, H), jnp.float32),
    mesh=_mesh,
    scratch_types=[
        pltpu.VMEM((NCHUNK, K), jnp.int32),
        pltpu.VMEM((K, 16), jnp.float32),
        pltpu.VMEM((RPS, 16), jnp.float32),
        pltpu.VMEM((RPS, H), jnp.float32),
        pltpu.VMEM_SHARED((NP, 16), jnp.float32),
        pltpu.SemaphoreType.DMA,
        pltpu.SemaphoreType.DMA,
    ],
    compiler_params=_sc_params,
)
def _sc_count(eidx_hbm, zeros_hbm, ones_hbm, out_hbm, didx, ones_v,
              buf16, buf64, acc, isem, ssem):
    cid = lax.axis_index("c")
    sid = lax.axis_index("s")
    wid = sid * NC + cid
    pltpu.async_copy(eidx_hbm.at[1].at[wid], didx, isem)
    pltpu.sync_copy(ones_hbm, ones_v)
    pltpu.sync_copy(zeros_hbm.at[pl.ds(sid * RPS, RPS)],
                    acc.at[pl.ds(sid * RPS, RPS)])
    pltpu.make_async_copy(eidx_hbm.at[1].at[wid], didx, isem).wait()
    plsc.subcore_barrier()

    @pl.loop(0, NCHUNK)
    def _(c):
        pltpu.async_copy(ones_v, acc.at[didx.at[c]], ssem, add=True)

    @pl.loop(0, NCHUNK)
    def _(c):
        pltpu.make_async_copy(ones_v, acc.at[didx.at[0]], ssem).wait()

    plsc.subcore_barrier()
    pltpu.sync_copy(acc.at[pl.ds(sid * RPS, RPS)], buf16)

    @pl.loop(0, RPS)
    def _(r):
        v = buf16[r]
        buf64[r, 0:16] = v
        buf64[r, 16:32] = v
        buf64[r, 32:48] = v
        buf64[r, 48:64] = v

    pltpu.sync_copy(buf64, out_hbm.at[pl.ds(cid * NP + sid * RPS, RPS)])


@functools.partial(
    pl.kernel,
    out_type=jax.ShapeDtypeStruct((NC * NP, H), jnp.float32),
    mesh=_mesh,
    scratch_types=[
        pltpu.VMEM((NCHUNK, K), jnp.int32),
        pltpu.VMEM((NCHUNK, K), jnp.int32),
        pltpu.VMEM((K, H), jnp.float32),
        pltpu.VMEM((K, H), jnp.float32),
        pltpu.VMEM((K, H), jnp.float32),
        pltpu.VMEM((K, H), jnp.float32),
        pltpu.VMEM_SHARED((NP, H), jnp.float32),
        pltpu.SemaphoreType.DMA,
        pltpu.SemaphoreType.DMA,
        pltpu.SemaphoreType.DMA,
        pltpu.SemaphoreType.DMA,
        pltpu.SemaphoreType.DMA,
        pltpu.SemaphoreType.DMA,
        pltpu.SemaphoreType.DMA,
        pltpu.SemaphoreType.DMA,
        pltpu.SemaphoreType.DMA,
    ],
    compiler_params=_sc_params,
)
def _sc_agg(hs_hbm, eidx_hbm, zeros_hbm, out_hbm,
            sidx, didx, rows0, rows1, rows2, rows3, acc,
            gsem0, gsem1, gsem2, gsem3, ssem0, ssem1, ssem2, ssem3, isem):
    cid = lax.axis_index("c")
    sid = lax.axis_index("s")
    wid = sid * NC + cid
    pltpu.async_copy(eidx_hbm.at[0].at[wid], sidx, isem)
    pltpu.async_copy(eidx_hbm.at[1].at[wid], didx, isem)
    pltpu.sync_copy(zeros_hbm.at[pl.ds(sid * RPS, RPS)],
                    acc.at[pl.ds(sid * RPS, RPS)])
    pltpu.make_async_copy(eidx_hbm.at[0].at[wid], sidx, isem).wait()
    pltpu.make_async_copy(eidx_hbm.at[1].at[wid], didx, isem).wait()
    plsc.subcore_barrier()

    rows = (rows0, rows1, rows2, rows3)
    gsem = (gsem0, gsem1, gsem2, gsem3)
    ssem = (ssem0, ssem1, ssem2, ssem3)

    def start_gather(c, b):
        pltpu.async_copy(hs_hbm.at[sidx.at[c]], rows[b], gsem[b])

    def wait_gather(c, b):
        pltpu.make_async_copy(hs_hbm.at[sidx.at[c]], rows[b], gsem[b]).wait()

    def start_scatter(c, b):
        pltpu.async_copy(rows[b], acc.at[didx.at[c]], ssem[b], add=True)

    def wait_scatter(c, b):
        pltpu.make_async_copy(rows[b], acc.at[didx.at[c]], ssem[b]).wait()

    start_gather(0, 0)
    start_gather(1, 1)
    start_gather(2, 2)
    wait_gather(0, 0)
    start_scatter(0, 0)
    start_gather(3, 3)

    @pl.loop(1, NCHUNK - 4, step=4)
    def _(c0):
        for j in range(4):
            c = c0 + j
            b = (1 + j) % 4
            wait_gather(c, b)
            start_scatter(c, b)
            wait_scatter(c - 1, (b + 3) % 4)
            start_gather(c + 3, (b + 3) % 4)

    c = NCHUNK - 4
    wait_gather(c, c % 4)
    start_scatter(c, c % 4)
    wait_scatter(c - 1, (c + 3) % 4)
    start_gather(c + 3, (c + 3) % 4)
    for c in range(NCHUNK - 3, NCHUNK):
        wait_gather(c, c % 4)
        start_scatter(c, c % 4)
        wait_scatter(c - 1, (c - 1) % 4)
    wait_scatter(NCHUNK - 1, (NCHUNK - 1) % 4)

    plsc.subcore_barrier()
    pltpu.sync_copy(acc.at[pl.ds(sid * RPS, RPS)],
                    out_hbm.at[pl.ds(cid * NP + sid * RPS, RPS)])


NPAIR = N // 2
NPP = NP // 2


def _tc_mm1_body(x_ref, w1b_ref, h_ref):
    h_ref[...] = jnp.dot(x_ref[...], w1b_ref[...],
                         preferred_element_type=jnp.float32)


_tc_mm1 = pl.pallas_call(
    _tc_mm1_body,
    out_shape=jax.ShapeDtypeStruct((NPAIR, 2 * H), jnp.float32),
)


def _tc_scale_body(h_ref, cnt_ref, hs_ref, dinv_ref):
    cnt = cnt_ref[0:NPAIR, :] + cnt_ref[NPP:NPP + NPAIR, :]
    dinv = lax.rsqrt(cnt + 1.0)
    hs_ref[...] = h_ref[...] * dinv
    dinv_ref[...] = dinv


_tc_scale = pl.pallas_call(
    _tc_scale_body,
    out_shape=(jax.ShapeDtypeStruct((NPAIR, 2 * H), jnp.float32),
               jax.ShapeDtypeStruct((NPAIR, 2 * H), jnp.float32)),
)


def _tc_mid_body(agg_ref, hs_ref, dinv_ref, b1_ref, w2b_ref, hs2_ref):
    h = agg_ref[0:NPAIR, :] + agg_ref[NPP:NPP + NPAIR, :] + hs_ref[...]
    h = jnp.maximum(dinv_ref[...] * h + b1_ref[...], 0.0)
    hs2_ref[...] = jnp.dot(h, w2b_ref[...],
                           preferred_element_type=jnp.float32) * dinv_ref[...]


_tc_mid = pl.pallas_call(
    _tc_mid_body,
    out_shape=jax.ShapeDtypeStruct((NPAIR, 2 * H), jnp.float32),
)


def _tc_final_body(agg_ref, hs_ref, dinv_ref, b2_ref, be_ref, bo_ref,
                   wlin_ref, blin_ref, out_ref):
    h = agg_ref[0:NPAIR, :] + agg_ref[NPP:NPP + NPAIR, :] + hs_ref[...]
    h = jnp.maximum(dinv_ref[...] * h + b2_ref[...], 0.0)
    gids = lax.broadcasted_iota(jnp.int32, (G, NPAIR), 0)
    me = (be_ref[...] == gids).astype(jnp.float32)
    mo = (bo_ref[...] == gids).astype(jnp.float32)
    sums = (jnp.dot(me, h[:, 0:H], preferred_element_type=jnp.float32)
            + jnp.dot(mo, h[:, H:2 * H], preferred_element_type=jnp.float32))
    counts = jnp.sum(me + mo, axis=1, keepdims=True)
    pooled = sums / jnp.maximum(counts, 1.0)
    logits = jnp.dot(pooled, wlin_ref[...],
                     preferred_element_type=jnp.float32) + blin_ref[...]
    z = logits - jnp.max(logits, axis=1, keepdims=True)
    out_ref[...] = z - jnp.log(jnp.sum(jnp.exp(z), axis=1, keepdims=True))


_tc_final = pl.pallas_call(
    _tc_final_body,
    out_shape=jax.ShapeDtypeStruct((G, C), jnp.float32),
)


def _blockdiag(w, k):
    z = jnp.zeros((k, w.shape[0], k, w.shape[1]), jnp.float32)
    z = z.at[jnp.arange(k), :, jnp.arange(k), :].set(w)
    return z.reshape(k * w.shape[0], k * w.shape[1])


def kernel(x, edge_index, batch, W1, b1, W2, b2, Wlin, blin):
    eidx = edge_index.reshape(2, NW, NCHUNK, K)
    zeros_nh = jnp.zeros((NP, H), jnp.float32)
    zeros_n16 = jnp.zeros((NP, 16), jnp.float32)
    ones_k16 = jnp.ones((K, 16), jnp.float32)
    w1b = _blockdiag(W1, 2)
    w2b = _blockdiag(W2, 2)
    b1p = jnp.concatenate([b1, b1]).reshape(1, 2 * H)
    b2p = jnp.concatenate([b2, b2]).reshape(1, 2 * H)
    be = batch[0::2].reshape(1, NPAIR)
    bo = batch[1::2].reshape(1, NPAIR)
    cnt2 = _sc_count(eidx, zeros_n16, ones_k16)
    h1p = _tc_mm1(x.reshape(NPAIR, 2 * F_IN), w1b)
    hs1p, dinv2p = _tc_scale(h1p, cnt2.reshape(NP, 2 * H))
    agg1 = _sc_agg(hs1p.reshape(N, H), eidx, zeros_nh)
    hs2p = _tc_mid(agg1.reshape(NP, 2 * H), hs1p, dinv2p, b1p, w2b)
    agg2 = _sc_agg(hs2p.reshape(N, H), eidx, zeros_nh)
    return _tc_final(agg2.reshape(NP, 2 * H), hs2p, dinv2p, b2p, be, bo,
                     Wlin, blin.reshape(1, C))


# speedup vs baseline: 60.4816x; 1.0402x over previous
import functools

import jax
import jax.numpy as jnp
from jax import lax
from jax.experimental import pallas as pl
from jax.experimental.pallas import tpu as pltpu
from jax.experimental.pallas import tpu_sc as plsc

N = 10000
E = 320000
F_IN = 128
H = 64
C = 6
G = 64

NC = 2
NS = 16
NW = NC * NS
K = 80
EPW = E // NW
NCHUNK = EPW // K
NP = 10240
RPS = NP // NS

_mesh = plsc.VectorSubcoreMesh(core_axis_name="c", subcore_axis_name="s")
_sc_params = pltpu.CompilerParams(use_tc_tiling_on_sc=False)


@functools.partial(
    pl.kernel,
    out_type=jax.ShapeDtypeStruct((NC * NP, H), jnp.float32),
    mesh=_mesh,
    scratch_types=[
        pltpu.VMEM((NCHUNK, K), jnp.int32),
        pltpu.VMEM((K, 16), jnp.float32),
        pltpu.VMEM((RPS, 16), jnp.float32),
        pltpu.VMEM((RPS, H), jnp.float32),
        pltpu.VMEM_SHARED((NP, 16), jnp.float32),
        pltpu.SemaphoreType.DMA,
        pltpu.SemaphoreType.DMA,
    ],
    compiler_params=_sc_params,
)
def _sc_count(eidx_hbm, out_hbm, didx, ones_v, buf16, buf64, acc,
              isem, ssem):
    cid = lax.axis_index("c")
    sid = lax.axis_index("s")
    wid = sid * NC + cid
    pltpu.async_copy(eidx_hbm.at[1].at[wid], didx, isem)
    one16 = jnp.full((16,), 1.0, jnp.float32)
    zero16 = jnp.zeros((16,), jnp.float32)

    @pl.loop(0, K)
    def _(r):
        ones_v[r, 0:16] = one16

    @pl.loop(0, RPS)
    def _(r):
        buf16[r, 0:16] = zero16

    pltpu.sync_copy(buf16, acc.at[pl.ds(sid * RPS, RPS)])
    pltpu.make_async_copy(eidx_hbm.at[1].at[wid], didx, isem).wait()
    plsc.subcore_barrier()

    @pl.loop(0, NCHUNK)
    def _(c):
        pltpu.async_copy(ones_v, acc.at[didx.at[c]], ssem, add=True)

    @pl.loop(0, NCHUNK)
    def _(c):
        pltpu.make_async_copy(ones_v, acc.at[didx.at[0]], ssem).wait()

    plsc.subcore_barrier()
    pltpu.sync_copy(acc.at[pl.ds(sid * RPS, RPS)], buf16)

    @pl.loop(0, RPS)
    def _(r):
        v = buf16[r]
        buf64[r, 0:16] = v
        buf64[r, 16:32] = v
        buf64[r, 32:48] = v
        buf64[r, 48:64] = v

    pltpu.sync_copy(buf64, out_hbm.at[pl.ds(cid * NP + sid * RPS, RPS)])


@functools.partial(
    pl.kernel,
    out_type=jax.ShapeDtypeStruct((NC * NP, H), jnp.float32),
    mesh=_mesh,
    scratch_types=[
        pltpu.VMEM((NCHUNK, K), jnp.int32),
        pltpu.VMEM((NCHUNK, K), jnp.int32),
        pltpu.VMEM((K, H), jnp.float32),
        pltpu.VMEM((K, H), jnp.float32),
        pltpu.VMEM((K, H), jnp.float32),
        pltpu.VMEM((K, H), jnp.float32),
        pltpu.VMEM_SHARED((NP, H), jnp.float32),
        pltpu.SemaphoreType.DMA,
        pltpu.SemaphoreType.DMA,
        pltpu.SemaphoreType.DMA,
        pltpu.SemaphoreType.DMA,
        pltpu.SemaphoreType.DMA,
        pltpu.SemaphoreType.DMA,
        pltpu.SemaphoreType.DMA,
        pltpu.SemaphoreType.DMA,
        pltpu.SemaphoreType.DMA,
    ],
    compiler_params=_sc_params,
)
def _sc_agg(hs_hbm, eidx_hbm, out_hbm,
            sidx, didx, rows0, rows1, rows2, rows3, acc,
            gsem0, gsem1, gsem2, gsem3, ssem0, ssem1, ssem2, ssem3, isem):
    cid = lax.axis_index("c")
    sid = lax.axis_index("s")
    wid = sid * NC + cid
    pltpu.async_copy(eidx_hbm.at[0].at[wid], sidx, isem)
    pltpu.async_copy(eidx_hbm.at[1].at[wid], didx, isem)
    rows = (rows0, rows1, rows2, rows3)
    zero16 = jnp.zeros((16,), jnp.float32)

    @pl.loop(0, K)
    def _(r):
        for _b in range(4):
            rows[_b][r, 0:16] = zero16
            rows[_b][r, 16:32] = zero16
            rows[_b][r, 32:48] = zero16
            rows[_b][r, 48:64] = zero16

    for _i in range(8):
        pltpu.async_copy(rows[_i % 4],
                         acc.at[pl.ds(sid * RPS + _i * K, K)], isem)
    for _i in range(8):
        pltpu.make_async_copy(rows[_i % 4],
                              acc.at[pl.ds(sid * RPS + _i * K, K)],
                              isem).wait()
    pltpu.make_async_copy(eidx_hbm.at[0].at[wid], sidx, isem).wait()
    pltpu.make_async_copy(eidx_hbm.at[1].at[wid], didx, isem).wait()
    plsc.subcore_barrier()

    gsem = (gsem0, gsem1, gsem2, gsem3)
    ssem = (ssem0, ssem1, ssem2, ssem3)

    def start_gather(c, b):
        pltpu.async_copy(hs_hbm.at[sidx.at[c]], rows[b], gsem[b])

    def wait_gather(c, b):
        pltpu.make_async_copy(hs_hbm.at[sidx.at[c]], rows[b], gsem[b]).wait()

    def start_scatter(c, b):
        pltpu.async_copy(rows[b], acc.at[didx.at[c]], ssem[b], add=True)

    def wait_scatter(c, b):
        pltpu.make_async_copy(rows[b], acc.at[didx.at[c]], ssem[b]).wait()

    start_gather(0, 0)
    start_gather(1, 1)
    start_gather(2, 2)
    wait_gather(0, 0)
    start_scatter(0, 0)
    start_gather(3, 3)

    @pl.loop(1, NCHUNK - 4, step=4)
    def _(c0):
        for j in range(4):
            c = c0 + j
            b = (1 + j) % 4
            wait_gather(c, b)
            start_scatter(c, b)
            wait_scatter(c - 1, (b + 3) % 4)
            start_gather(c + 3, (b + 3) % 4)

    c = NCHUNK - 4
    wait_gather(c, c % 4)
    start_scatter(c, c % 4)
    wait_scatter(c - 1, (c + 3) % 4)
    start_gather(c + 3, (c + 3) % 4)
    for c in range(NCHUNK - 3, NCHUNK):
        wait_gather(c, c % 4)
        start_scatter(c, c % 4)
        wait_scatter(c - 1, (c - 1) % 4)
    wait_scatter(NCHUNK - 1, (NCHUNK - 1) % 4)

    plsc.subcore_barrier()
    pltpu.sync_copy(acc.at[pl.ds(sid * RPS, RPS)],
                    out_hbm.at[pl.ds(cid * NP + sid * RPS, RPS)])


NPAIR = N // 2
NPP = NP // 2


def _tc_mm1_body(x_ref, w1b_ref, h_ref):
    h_ref[...] = jnp.dot(x_ref[...], w1b_ref[...],
                         preferred_element_type=jnp.float32)


_tc_mm1 = pl.pallas_call(
    _tc_mm1_body,
    out_shape=jax.ShapeDtypeStruct((NPAIR, 2 * H), jnp.float32),
)


def _tc_scale_body(h_ref, cnt_ref, hs_ref, dinv_ref):
    cnt = cnt_ref[0:NPAIR, :] + cnt_ref[NPP:NPP + NPAIR, :]
    dinv = lax.rsqrt(cnt + 1.0)
    hs_ref[...] = h_ref[...] * dinv
    dinv_ref[...] = dinv


_tc_scale = pl.pallas_call(
    _tc_scale_body,
    out_shape=(jax.ShapeDtypeStruct((NPAIR, 2 * H), jnp.float32),
               jax.ShapeDtypeStruct((NPAIR, 2 * H), jnp.float32)),
)


def _tc_mid_body(agg_ref, hs_ref, dinv_ref, b1_ref, w2b_ref, hs2_ref):
    h = agg_ref[0:NPAIR, :] + agg_ref[NPP:NPP + NPAIR, :] + hs_ref[...]
    h = jnp.maximum(dinv_ref[...] * h + b1_ref[...], 0.0)
    hs2_ref[...] = jnp.dot(h, w2b_ref[...],
                           preferred_element_type=jnp.float32) * dinv_ref[...]


_tc_mid = pl.pallas_call(
    _tc_mid_body,
    out_shape=jax.ShapeDtypeStruct((NPAIR, 2 * H), jnp.float32),
)


def _tc_final_body(agg_ref, hs_ref, dinv_ref, b2_ref, be_ref, bo_ref,
                   wlin_ref, blin_ref, out_ref):
    h = agg_ref[0:NPAIR, :] + agg_ref[NPP:NPP + NPAIR, :] + hs_ref[...]
    h = jnp.maximum(dinv_ref[...] * h + b2_ref[...], 0.0)
    gids = lax.broadcasted_iota(jnp.int32, (G, NPAIR), 0)
    me = (be_ref[...] == gids).astype(jnp.float32)
    mo = (bo_ref[...] == gids).astype(jnp.float32)
    sums = (jnp.dot(me, h[:, 0:H], preferred_element_type=jnp.float32)
            + jnp.dot(mo, h[:, H:2 * H], preferred_element_type=jnp.float32))
    counts = jnp.sum(me + mo, axis=1, keepdims=True)
    pooled = sums / jnp.maximum(counts, 1.0)
    logits = jnp.dot(pooled, wlin_ref[...],
                     preferred_element_type=jnp.float32) + blin_ref[...]
    z = logits - jnp.max(logits, axis=1, keepdims=True)
    out_ref[...] = z - jnp.log(jnp.sum(jnp.exp(z), axis=1, keepdims=True))


_tc_final = pl.pallas_call(
    _tc_final_body,
    out_shape=jax.ShapeDtypeStruct((G, C), jnp.float32),
)


def _blockdiag(w, k):
    z = jnp.zeros((k, w.shape[0], k, w.shape[1]), jnp.float32)
    z = z.at[jnp.arange(k), :, jnp.arange(k), :].set(w)
    return z.reshape(k * w.shape[0], k * w.shape[1])


def kernel(x, edge_index, batch, W1, b1, W2, b2, Wlin, blin):
    eidx = edge_index.reshape(2, NW, NCHUNK, K)
    w1b = _blockdiag(W1, 2)
    w2b = _blockdiag(W2, 2)
    b1p = jnp.concatenate([b1, b1]).reshape(1, 2 * H)
    b2p = jnp.concatenate([b2, b2]).reshape(1, 2 * H)
    be = batch[0::2].reshape(1, NPAIR)
    bo = batch[1::2].reshape(1, NPAIR)
    cnt2 = _sc_count(eidx)
    h1p = _tc_mm1(x.reshape(NPAIR, 2 * F_IN), w1b)
    hs1p, dinv2p = _tc_scale(h1p, cnt2.reshape(NP, 2 * H))
    agg1 = _sc_agg(hs1p.reshape(N, H), eidx)
    hs2p = _tc_mid(agg1.reshape(NP, 2 * H), hs1p, dinv2p, b1p, w2b)
    agg2 = _sc_agg(hs2p.reshape(N, H), eidx)
    return _tc_final(agg2.reshape(NP, 2 * H), hs2p, dinv2p, b2p, be, bo,
                     Wlin, blin.reshape(1, C))
